# baseline (speedup 1.0000x reference)
import functools

import jax
import jax.numpy as jnp
from jax import lax
from jax.experimental import pallas as pl
from jax.experimental.pallas import tpu as pltpu
from jax.experimental.pallas import tpu_sc as plsc

_N = 8192 * 1024
_B = 4
_NC = 2
_NS = 16
_NW = _NC * _NS
_R = _N // 4
_PER_W = _N // _NW

_CHS = 2048
_ROWS_W = _R // _NW
_NCHS = _ROWS_W // _CHS

_CH = 2048
_NCH = _PER_W // _CH
_G = _CH // 16

_SC_PARAMS = pltpu.CompilerParams(use_tc_tiling_on_sc=False,
                                  needs_layout_passes=False)
_TC_PARAMS = pltpu.CompilerParams(use_tc_tiling_on_sc=True,
                                  needs_layout_passes=False)

_XROWS_W = 8192 // _NW
_XR_CH = 8
_WORDS_CH = _XR_CH * 1024 * _B
_WORDS_W = _XROWS_W * 1024 * _B


def _build_table(x):
    mesh = plsc.VectorSubcoreMesh(core_axis_name="c", subcore_axis_name="s")

    @functools.partial(
        pl.kernel,
        mesh=mesh,
        out_type=jax.ShapeDtypeStruct((4 * _N,), jnp.float32),
        scratch_types=[
            pltpu.VMEM((2, _B, _XR_CH, 1024), jnp.float32),
            pltpu.VMEM((_WORDS_CH,), jnp.float32),
            pltpu.VMEM((_WORDS_CH,), jnp.float32),
            pltpu.SemaphoreType.DMA,
            pltpu.SemaphoreType.DMA,
        ],
        compiler_params=_TC_PARAMS,
    )
    def k(x_hbm, tab_hbm, in_v, row_v0, row_v1, sem_g, sem_s):
        wid = lax.axis_index("s") * _NC + lax.axis_index("c")
        xr0 = wid * _XROWS_W
        w0 = wid * _WORDS_W
        iota = lax.iota(jnp.int32, 16)
        wpat = (lax.shift_right_logical(iota, 2) * 16
                + lax.shift_left(iota & 3, 2))
        row_v = (row_v0, row_v1)

        def loads(c, par):
            return [
                pltpu.make_async_copy(
                    x_hbm.at[b].at[pl.ds(xr0 + c * _XR_CH, _XR_CH)],
                    in_v.at[par].at[b], sem_g)
                for b in range(_B)
            ]

        def store(c, par):
            return pltpu.make_async_copy(
                row_v[par],
                tab_hbm.at[pl.ds(w0 + c * _WORDS_CH, _WORDS_CH)], sem_s)

        def interleave(par):
            for b in range(_B):
                @plsc.parallel_loop(0, _XR_CH * 64, unroll=16)
                def _(g2):
                    i = lax.shift_right_logical(g2, 6)
                    g = g2 & 63
                    vals = in_v[par, b, i, pl.ds(g * 16, 16)]
                    plsc.store_scatter(
                        row_v[par],
                        [i * 4096 + b + g * 64 + wpat], vals)

        for cp in loads(0, 0):
            cp.start()

        @pl.loop(0, _NCHS // 2)
        def _(jj):
            for p in (0, 1):
                c = 2 * jj + p
                for cp in loads(c, p):
                    cp.wait()

                @pl.when(c + 1 < _NCHS)
                def _():
                    for cp in loads(c + 1, 1 - p):
                        cp.start()

                @pl.when(c >= 2)
                def _():
                    store(c - 2, p).wait()

                interleave(p)
                store(c, p).start()

        for c in (_NCHS - 2, _NCHS - 1):
            store(c, c % 2).wait()

    return k(x)


def _gather(table, perm):
    mesh = plsc.VectorSubcoreMesh(core_axis_name="c", subcore_axis_name="s")

    @functools.partial(
        pl.kernel,
        mesh=mesh,
        out_type=jax.ShapeDtypeStruct((_B, _N), jnp.float32),
        scratch_types=[
            pltpu.VMEM((2, _CH), jnp.int32),
            pltpu.VMEM((2, _CH), jnp.int32),
            pltpu.VMEM((2, _CH, 16), jnp.float32),
            pltpu.VMEM((2, _B, _CH), jnp.float32),
            pltpu.SemaphoreType.DMA,
            pltpu.SemaphoreType.DMA,
            pltpu.SemaphoreType.DMA,
        ],
        compiler_params=_SC_PARAMS,
    )
    def k(table_hbm, perm_hbm, out_hbm, pidx_v, gidx_v, val_v, outb_v,
          sem_g, sem_s, sem_p):
        wid = lax.axis_index("s") * _NC + lax.axis_index("c")
        w_base = wid * _PER_W
        iota = lax.iota(jnp.int32, 16)

        def base(c):
            return w_base + c * _CH

        def load_perm(c, par):
            return pltpu.make_async_copy(perm_hbm.at[pl.ds(base(c), _CH)],
                                         pidx_v.at[par], sem_p)

        def compute_gidx(par):
            @plsc.parallel_loop(0, _G, unroll=8)
            def _(g):
                s = g * 16
                pvec = pidx_v[par, pl.ds(s, 16)]
                gidx_v[par, pl.ds(s, 16)] = lax.shift_right_logical(pvec, 2)

        def gather(par):
            return pltpu.make_async_copy(table_hbm.at[gidx_v.at[par]],
                                         val_v.at[par], sem_g)

        def extract(par):
            @plsc.parallel_loop(0, _G, unroll=4)
            def _(g):
                s = g * 16
                pvec = pidx_v[par, pl.ds(s, 16)]
                col0 = lax.shift_left(pvec & 3, 2)
                rows = s + iota
                for b in range(_B):
                    outb_v[par, b, pl.ds(s, 16)] = plsc.load_gather(
                        val_v.at[par], [rows, col0 + b])

        def scatter(c, par, b):
            return pltpu.make_async_copy(
                outb_v.at[par].at[b],
                out_hbm.at[b].at[pl.ds(base(c), _CH)], sem_s)

        load_perm(0, 0).start()
        load_perm(0, 0).wait()
        compute_gidx(0)
        gather(0).start()
        load_perm(1, 1).start()

        @pl.loop(0, _NCH // 2)
        def _(j):
            for p in (0, 1):
                c = 2 * j + p
                @pl.when(c + 1 < _NCH)
                def _():
                    load_perm(c + 1, 1 - p).wait()

                compute_gidx(1 - p)

                @pl.when(c + 1 < _NCH)
                def _():
                    gather(1 - p).start()

                gather(p).wait()

                @pl.when(c >= 2)
                def _():
                    for b in range(_B):
                        scatter(c - 2, p, b).wait()

                extract(p)
                for b in range(_B):
                    scatter(c, p, b).start()

                @pl.when(c + 2 < _NCH)
                def _():
                    load_perm(c + 2, p).start()

        for c in (_NCH - 2, _NCH - 1):
            for b in range(_B):
                scatter(c, c % 2, b).wait()

    return k(table, perm)


def kernel(x, perm):
    bsz = x.shape[0]
    dims = x.shape[1:]
    table = _build_table(x).reshape(_R, 16)
    out = _gather(table, perm)
    return out.reshape((bsz,) + dims)


# speedup vs baseline: 1.0676x; 1.0676x over previous
import functools

import jax
import jax.numpy as jnp
from jax import lax
from jax.experimental import pallas as pl
from jax.experimental.pallas import tpu as pltpu
from jax.experimental.pallas import tpu_sc as plsc

_N = 8192 * 1024
_B = 4
_NC = 2
_NS = 16
_NW = _NC * _NS
_R = _N // 4
_PER_W = _N // _NW

_CHS = 2048
_ROWS_W = _R // _NW
_NCHS = _ROWS_W // _CHS

_CH = 1024
_NCH = _PER_W // _CH
_G = _CH // 16

_SC_PARAMS = pltpu.CompilerParams(use_tc_tiling_on_sc=False,
                                  needs_layout_passes=False)
_TC_PARAMS = pltpu.CompilerParams(use_tc_tiling_on_sc=True,
                                  needs_layout_passes=False)

_XROWS_W = 8192 // _NW
_XR_CH = 8
_WORDS_CH = _XR_CH * 1024 * _B
_WORDS_W = _XROWS_W * 1024 * _B


def _build_table(x):
    mesh = plsc.VectorSubcoreMesh(core_axis_name="c", subcore_axis_name="s")

    @functools.partial(
        pl.kernel,
        mesh=mesh,
        out_type=jax.ShapeDtypeStruct((4 * _N,), jnp.float32),
        scratch_types=[
            pltpu.VMEM((2, _B, _XR_CH, 1024), jnp.float32),
            pltpu.VMEM((_WORDS_CH,), jnp.float32),
            pltpu.VMEM((_WORDS_CH,), jnp.float32),
            pltpu.SemaphoreType.DMA,
            pltpu.SemaphoreType.DMA,
        ],
        compiler_params=_TC_PARAMS,
    )
    def k(x_hbm, tab_hbm, in_v, row_v0, row_v1, sem_g, sem_s):
        wid = lax.axis_index("s") * _NC + lax.axis_index("c")
        xr0 = wid * _XROWS_W
        w0 = wid * _WORDS_W
        iota = lax.iota(jnp.int32, 16)
        wpat = (lax.shift_right_logical(iota, 2) * 16
                + lax.shift_left(iota & 3, 2))
        row_v = (row_v0, row_v1)

        def loads(c, par):
            return [
                pltpu.make_async_copy(
                    x_hbm.at[b].at[pl.ds(xr0 + c * _XR_CH, _XR_CH)],
                    in_v.at[par].at[b], sem_g)
                for b in range(_B)
            ]

        def store(c, par):
            return pltpu.make_async_copy(
                row_v[par],
                tab_hbm.at[pl.ds(w0 + c * _WORDS_CH, _WORDS_CH)], sem_s)

        def interleave(par):
            for b in range(_B):
                @plsc.parallel_loop(0, _XR_CH * 64, unroll=8)
                def _(g2):
                    i = lax.shift_right_logical(g2, 6)
                    g = g2 & 63
                    vals = in_v[par, b, i, pl.ds(g * 16, 16)]
                    plsc.store_scatter(
                        row_v[par],
                        [i * 4096 + b + g * 64 + wpat], vals)

        for cp in loads(0, 0):
            cp.start()

        @pl.loop(0, _NCHS // 2)
        def _(jj):
            for p in (0, 1):
                c = 2 * jj + p
                for cp in loads(c, p):
                    cp.wait()

                @pl.when(c + 1 < _NCHS)
                def _():
                    for cp in loads(c + 1, 1 - p):
                        cp.start()

                @pl.when(c >= 2)
                def _():
                    store(c - 2, p).wait()

                interleave(p)
                store(c, p).start()

        for c in (_NCHS - 2, _NCHS - 1):
            store(c, c % 2).wait()

    return k(x)


def _gather(table, perm):
    mesh = plsc.VectorSubcoreMesh(core_axis_name="c", subcore_axis_name="s")

    @functools.partial(
        pl.kernel,
        mesh=mesh,
        out_type=jax.ShapeDtypeStruct((_B, _N), jnp.float32),
        scratch_types=[
            pltpu.VMEM((4, _CH), jnp.int32),
            pltpu.VMEM((4, _CH), jnp.int32),
            pltpu.VMEM((4, _CH, 16), jnp.float32),
            pltpu.VMEM((2, _B, _CH), jnp.float32),
            pltpu.SemaphoreType.DMA,
            pltpu.SemaphoreType.DMA,
            pltpu.SemaphoreType.DMA,
        ],
        compiler_params=_SC_PARAMS,
    )
    def k(table_hbm, perm_hbm, out_hbm, pidx_v, gidx_v, val_v, outb_v,
          sem_g, sem_s, sem_p):
        wid = lax.axis_index("s") * _NC + lax.axis_index("c")
        w_base = wid * _PER_W
        iota = lax.iota(jnp.int32, 16)

        def base(c):
            return w_base + c * _CH

        def load_perm(c, par):
            return pltpu.make_async_copy(perm_hbm.at[pl.ds(base(c), _CH)],
                                         pidx_v.at[par], sem_p)

        def compute_gidx(par):
            @plsc.parallel_loop(0, _G, unroll=8)
            def _(g):
                s = g * 16
                pvec = pidx_v[par, pl.ds(s, 16)]
                gidx_v[par, pl.ds(s, 16)] = lax.shift_right_logical(pvec, 2)

        def gather(par):
            return pltpu.make_async_copy(table_hbm.at[gidx_v.at[par]],
                                         val_v.at[par], sem_g)

        def extract(par4, par2):
            @plsc.parallel_loop(0, _G, unroll=4)
            def _(g):
                s = g * 16
                pvec = pidx_v[par4, pl.ds(s, 16)]
                col0 = lax.shift_left(pvec & 3, 2)
                rows = s + iota
                for b in range(_B):
                    outb_v[par2, b, pl.ds(s, 16)] = plsc.load_gather(
                        val_v.at[par4], [rows, col0 + b])

        def scatter(c, par2, b):
            return pltpu.make_async_copy(
                outb_v.at[par2].at[b],
                out_hbm.at[b].at[pl.ds(base(c), _CH)], sem_s)

        for c0 in range(4):
            load_perm(c0, c0).start()
        for c0 in range(2):
            load_perm(c0, c0).wait()
            compute_gidx(c0)
            gather(c0).start()

        @pl.loop(0, _NCH // 4)
        def _(j):
            for p in (0, 1, 2, 3):
                c = 4 * j + p
                p2 = (p + 2) % 4

                @pl.when(c + 2 < _NCH)
                def _():
                    load_perm(c + 2, p2).wait()

                compute_gidx(p2)

                @pl.when(c + 2 < _NCH)
                def _():
                    gather(p2).start()

                gather(p).wait()

                @pl.when(c >= 2)
                def _():
                    for b in range(_B):
                        scatter(c - 2, p % 2, b).wait()

                extract(p, p % 2)
                for b in range(_B):
                    scatter(c, p % 2, b).start()

                @pl.when(c + 4 < _NCH)
                def _():
                    load_perm(c + 4, p).start()

        for c in (_NCH - 2, _NCH - 1):
            for b in range(_B):
                scatter(c, c % 2, b).wait()

    return k(table, perm)


def kernel(x, perm):
    bsz = x.shape[0]
    dims = x.shape[1:]
    table = _build_table(x).reshape(_R, 16)
    out = _gather(table, perm)
    return out.reshape((bsz,) + dims)
